# baseline (speedup 1.0000x reference)
import functools

import jax
import jax.numpy as jnp
import numpy as np
from jax import lax
from jax.experimental import pallas as pl
from jax.experimental.pallas import tpu as pltpu
from jax.experimental.pallas import tpu_sc as plsc

TOP_K = 8
NUM_TOKENS = 32768
NUM_EXPERTS = 64
TWO_EPS = np.float32(0.4)
NEG_TWO_EPS = np.float32(-0.4)

NUM_CORES = 2
NUM_SUBCORES = 16
NUM_WORKERS = NUM_CORES * NUM_SUBCORES
TPW = NUM_TOKENS // NUM_WORKERS
GROUPS = TPW // 16
STRIDE = NUM_EXPERTS + 1
BUF_WORDS = TPW * STRIDE
STAGE_OFF = BUF_WORDS - TPW * NUM_EXPERTS


def _routing_body(x_hbm, oi_hbm, ov_hbm, buf, oi_v, ov_v):
    wid = lax.axis_index("c") * NUM_SUBCORES + lax.axis_index("s")
    tok0 = wid * TPW

    lanes = lax.broadcasted_iota(jnp.int32, (16,), 0)
    neg_inf = jnp.full((16,), -jnp.inf, jnp.float32)
    f32_zero = jnp.zeros((16,), jnp.float32)
    i32_zero = jnp.zeros((16,), jnp.int32)

    pltpu.sync_copy(
        x_hbm.at[pl.ds(tok0 * NUM_EXPERTS, TPW * NUM_EXPERTS)],
        buf.at[pl.ds(STAGE_OFF, TPW * NUM_EXPERTS)],
    )

    @pl.loop(0, TPW, unroll=4)
    def _relayout(t):
        src = STAGE_OFF + t * NUM_EXPERTS
        dst = t * STRIDE
        vs = [buf[pl.ds(src + 16 * c, 16)] for c in range(4)]
        for c in range(4):
            plsc.store_scatter(buf, [lanes + (dst + 16 * c)], vs[c])

    @pl.loop(0, GROUPS)
    def _group(g):
        rowoff = lanes * STRIDE + g * (16 * STRIDE)
        tok = lanes + g * 16

        def _max_pass(e, carry):
            m, idx = carry
            v = plsc.load_gather(buf, [rowoff + e])
            gt = v > m
            return jnp.where(gt, v, m), jnp.where(gt, e, idx)

        m, idx = lax.fori_loop(0, NUM_EXPERTS, _max_pass,
                               (neg_inf, i32_zero), unroll=8)

        one = jnp.full((16,), 1.0, jnp.float32)
        for i in range(TOP_K):
            plsc.store_scatter(buf, [rowoff + idx], neg_inf)

            def _fused_pass(e, carry, m=m):
                denom, m2, idx2 = carry
                v = plsc.load_gather(buf, [rowoff + e])
                gap = v - m
                neg_thresh = NEG_TWO_EPS * jnp.maximum(jnp.abs(v), m)
                drop = gap < neg_thresh
                term = jnp.where(drop, f32_zero, jnp.exp(gap))
                gt = v > m2
                return (denom + term,
                        jnp.where(gt, v, m2),
                        jnp.where(gt, e, idx2))

            denom, m2, idx2 = lax.fori_loop(
                0, NUM_EXPERTS, _fused_pass,
                (one, neg_inf, i32_zero), unroll=8)

            orow = jnp.full((16,), g, jnp.int32)
            ocol = lanes * TOP_K + i
            plsc.store_scatter(oi_v, [orow, ocol], idx)
            plsc.store_scatter(ov_v, [orow, ocol], jnp.float32(1.0) / denom)
            m, idx = m2, idx2

    pltpu.sync_copy(oi_v, oi_hbm.at[pl.ds(wid * GROUPS, GROUPS)])
    pltpu.sync_copy(ov_v, ov_hbm.at[pl.ds(wid * GROUPS, GROUPS)])


@jax.jit
def kernel(router_logits):
    x = router_logits.astype(jnp.float32).reshape(-1)
    mesh = plsc.VectorSubcoreMesh(
        core_axis_name="c", subcore_axis_name="s",
        num_cores=NUM_CORES, num_subcores=NUM_SUBCORES)
    run = pl.kernel(
        _routing_body,
        out_type=[
            jax.ShapeDtypeStruct((NUM_TOKENS // 16, 16 * TOP_K), jnp.int32),
            jax.ShapeDtypeStruct((NUM_TOKENS // 16, 16 * TOP_K), jnp.float32),
        ],
        mesh=mesh,
        compiler_params=pltpu.CompilerParams(needs_layout_passes=False),
        scratch_types=[
            pltpu.VMEM((BUF_WORDS,), jnp.float32),
            pltpu.VMEM((GROUPS, 16 * TOP_K), jnp.int32),
            pltpu.VMEM((GROUPS, 16 * TOP_K), jnp.float32),
        ],
    )
    idxs, vals = run(x)
    return (idxs.reshape(NUM_TOKENS, TOP_K),
            vals.reshape(NUM_TOKENS, TOP_K))


# speedup vs baseline: 1.0374x; 1.0374x over previous
import functools

import jax
import jax.numpy as jnp
import numpy as np
from jax import lax
from jax.experimental import pallas as pl
from jax.experimental.pallas import tpu as pltpu
from jax.experimental.pallas import tpu_sc as plsc

TOP_K = 8
NUM_TOKENS = 32768
NUM_EXPERTS = 64
TWO_EPS = np.float32(0.4)
NEG_TWO_EPS = np.float32(-0.4)

NUM_CORES = 2
NUM_SUBCORES = 16
NUM_WORKERS = NUM_CORES * NUM_SUBCORES
TPW = NUM_TOKENS // NUM_WORKERS
GROUPS = TPW // 16
STRIDE = NUM_EXPERTS + 1
BUF_WORDS = TPW * STRIDE
STAGE_OFF = BUF_WORDS - TPW * NUM_EXPERTS


def _routing_body(x_hbm, oi_hbm, ov_hbm, buf, oi_v, ov_v):
    wid = lax.axis_index("c") * NUM_SUBCORES + lax.axis_index("s")
    tok0 = wid * TPW

    lanes = lax.broadcasted_iota(jnp.int32, (16,), 0)
    neg_inf = jnp.full((16,), -jnp.inf, jnp.float32)
    f32_zero = jnp.zeros((16,), jnp.float32)
    i32_zero = jnp.zeros((16,), jnp.int32)

    pltpu.sync_copy(
        x_hbm.at[pl.ds(tok0 * NUM_EXPERTS, TPW * NUM_EXPERTS)],
        buf.at[pl.ds(STAGE_OFF, TPW * NUM_EXPERTS)],
    )

    @pl.loop(0, TPW, unroll=4)
    def _relayout(t):
        src = STAGE_OFF + t * NUM_EXPERTS
        dst = t * STRIDE
        vs = [buf[pl.ds(src + 16 * c, 16)] for c in range(4)]
        for c in range(4):
            plsc.store_scatter(buf, [lanes + (dst + 16 * c)], vs[c])

    @pl.loop(0, GROUPS // 8)
    def _chunk(chunk):
      @pl.loop(0, 8)
      def _group(gi):
        g = gi + chunk * 8
        rowoff = lanes * STRIDE + g * (16 * STRIDE)

        def _max_pass(e, carry):
            m, idx = carry
            v = plsc.load_gather(buf, [rowoff + e])
            gt = v > m
            return jnp.where(gt, v, m), jnp.where(gt, e, idx)

        m, idx = lax.fori_loop(0, NUM_EXPERTS, _max_pass,
                               (neg_inf, i32_zero), unroll=8)

        one = jnp.full((16,), 1.0, jnp.float32)
        for i in range(TOP_K):
            plsc.store_scatter(buf, [rowoff + idx], neg_inf)

            def _fused_pass(e, carry, m=m):
                denom, m2, idx2 = carry
                v = plsc.load_gather(buf, [rowoff + e])
                gap = v - m
                neg_thresh = NEG_TWO_EPS * jnp.maximum(jnp.abs(v), m)
                drop = gap < neg_thresh
                term = jnp.where(drop, f32_zero, jnp.exp(gap))
                gt = v > m2
                return (denom + term,
                        jnp.where(gt, v, m2),
                        jnp.where(gt, e, idx2))

            denom, m2, idx2 = lax.fori_loop(
                0, NUM_EXPERTS, _fused_pass,
                (one, neg_inf, i32_zero), unroll=8)

            orow = gi * 16 + lanes
            ocol = jnp.full((16,), i, jnp.int32)
            plsc.store_scatter(oi_v, [orow, ocol], idx)
            plsc.store_scatter(ov_v, [orow, ocol], jnp.float32(1.0) / denom)
            m, idx = m2, idx2

      base = tok0 + chunk * 128
      pltpu.sync_copy(oi_v, oi_hbm.at[pl.ds(base, 128)])
      pltpu.sync_copy(ov_v, ov_hbm.at[pl.ds(base, 128)])


@jax.jit
def kernel(router_logits):
    x = router_logits.astype(jnp.float32).reshape(-1)
    mesh = plsc.VectorSubcoreMesh(
        core_axis_name="c", subcore_axis_name="s",
        num_cores=NUM_CORES, num_subcores=NUM_SUBCORES)
    run = pl.kernel(
        _routing_body,
        out_type=[
            jax.ShapeDtypeStruct((NUM_TOKENS, TOP_K), jnp.int32),
            jax.ShapeDtypeStruct((NUM_TOKENS, TOP_K), jnp.float32),
        ],
        mesh=mesh,
        compiler_params=pltpu.CompilerParams(needs_layout_passes=False),
        scratch_types=[
            pltpu.VMEM((BUF_WORDS,), jnp.float32),
            pltpu.VMEM((128, TOP_K), jnp.int32),
            pltpu.VMEM((128, TOP_K), jnp.float32),
        ],
    )
    idxs, vals = run(x)
    return idxs, vals


# speedup vs baseline: 1.0384x; 1.0010x over previous
import functools

import jax
import jax.numpy as jnp
import numpy as np
from jax import lax
from jax.experimental import pallas as pl
from jax.experimental.pallas import tpu as pltpu
from jax.experimental.pallas import tpu_sc as plsc

TOP_K = 8
NUM_TOKENS = 32768
NUM_EXPERTS = 64
TWO_EPS = np.float32(0.4)
NEG_TWO_EPS = np.float32(-0.4)

NUM_CORES = 2
NUM_SUBCORES = 16
NUM_WORKERS = NUM_CORES * NUM_SUBCORES
TPW = NUM_TOKENS // NUM_WORKERS
GROUPS = TPW // 16
STRIDE = NUM_EXPERTS + 1
BUF_WORDS = TPW * STRIDE
STAGE_OFF = BUF_WORDS - TPW * NUM_EXPERTS


def _routing_body(x_hbm, oi_hbm, ov_hbm, buf, oi_v, ov_v):
    wid = lax.axis_index("c") * NUM_SUBCORES + lax.axis_index("s")
    tok0 = wid * TPW

    lanes = lax.broadcasted_iota(jnp.int32, (16,), 0)
    neg_inf = jnp.full((16,), -jnp.inf, jnp.float32)
    f32_zero = jnp.zeros((16,), jnp.float32)
    i32_zero = jnp.zeros((16,), jnp.int32)

    pltpu.sync_copy(
        x_hbm.at[pl.ds(tok0 * NUM_EXPERTS, TPW * NUM_EXPERTS)],
        buf.at[pl.ds(STAGE_OFF, TPW * NUM_EXPERTS)],
    )

    @pl.loop(0, TPW, unroll=4)
    def _relayout(t):
        src = STAGE_OFF + t * NUM_EXPERTS
        dst = t * STRIDE
        vs = [buf[pl.ds(src + 16 * c, 16)] for c in range(4)]
        for c in range(4):
            plsc.store_scatter(buf, [lanes + (dst + 16 * c)], vs[c])

    @pl.loop(0, GROUPS // 8)
    def _chunk(chunk):
      @pl.loop(0, 8)
      def _group(gi):
        g = gi + chunk * 8
        rowoff = lanes * STRIDE + g * (16 * STRIDE)

        def _max_pass(e, carry):
            m, idx = carry
            v = plsc.load_gather(buf, [rowoff + e])
            gt = v > m
            return jnp.where(gt, v, m), jnp.where(gt, e, idx)

        m, idx = lax.fori_loop(0, NUM_EXPERTS, _max_pass,
                               (neg_inf, i32_zero), unroll=8)

        one = jnp.full((16,), 1.0, jnp.float32)
        for i in range(TOP_K):
            plsc.store_scatter(buf, [rowoff + idx], neg_inf)

            def _fused_pass(e, carry, m=m):
                denom, m2, idx2 = carry
                v = plsc.load_gather(buf, [rowoff + e])
                gap = v - m
                neg_thresh = NEG_TWO_EPS * jnp.maximum(jnp.abs(v), m)
                drop = gap < neg_thresh
                term = jnp.where(drop, f32_zero, jnp.exp(gap))
                gt = v > m2
                return (denom + term,
                        jnp.where(gt, v, m2),
                        jnp.where(gt, e, idx2))

            denom, m2, idx2 = lax.fori_loop(
                0, NUM_EXPERTS, _fused_pass,
                (one, neg_inf, i32_zero), unroll=8)

            orow = gi * 16 + lanes
            ocol = jnp.full((16,), i, jnp.int32)
            plsc.store_scatter(oi_v, [orow, ocol], idx)
            plsc.store_scatter(ov_v, [orow, ocol], jnp.float32(1.0) / denom)
            m, idx = m2, idx2

      base = tok0 + chunk * 128
      pltpu.sync_copy(oi_v, oi_hbm.at[pl.ds(base, 128)])
      pltpu.sync_copy(ov_v, ov_hbm.at[pl.ds(base, 128)])


@jax.jit
def kernel(router_logits):
    x = router_logits.astype(jnp.float32).reshape(-1)
    mesh = plsc.VectorSubcoreMesh(
        core_axis_name="c", subcore_axis_name="s",
        num_cores=NUM_CORES, num_subcores=NUM_SUBCORES)
    run = pl.kernel(
        _routing_body,
        out_type=[
            jax.ShapeDtypeStruct((NUM_TOKENS, TOP_K), jnp.int32),
            jax.ShapeDtypeStruct((NUM_TOKENS, TOP_K), jnp.float32),
        ],
        mesh=mesh,
        compiler_params=pltpu.CompilerParams(
            needs_layout_passes=False, use_tc_tiling_on_sc=True),
        scratch_types=[
            pltpu.VMEM((BUF_WORDS,), jnp.float32),
            pltpu.VMEM((128, TOP_K), jnp.int32),
            pltpu.VMEM((128, TOP_K), jnp.float32),
        ],
    )
    idxs, vals = run(x)
    return idxs, vals


# speedup vs baseline: 1.0673x; 1.0278x over previous
import functools

import jax
import jax.numpy as jnp
import numpy as np
from jax import lax
from jax.experimental import pallas as pl
from jax.experimental.pallas import tpu as pltpu
from jax.experimental.pallas import tpu_sc as plsc

TOP_K = 8
NUM_TOKENS = 32768
NUM_EXPERTS = 64
TWO_EPS = np.float32(0.4)
NEG_TWO_EPS = np.float32(-0.4)

NUM_CORES = 2
NUM_SUBCORES = 16
NUM_WORKERS = NUM_CORES * NUM_SUBCORES
TPW = NUM_TOKENS // NUM_WORKERS
GROUPS = TPW // 16
STRIDE = NUM_EXPERTS + 1
BUF_WORDS = TPW * STRIDE
STAGE_OFF = BUF_WORDS - TPW * NUM_EXPERTS


def _routing_body(x_hbm, oi_hbm, ov_hbm, buf, stage, oi_v, ov_v):
    wid = lax.axis_index("c") * NUM_SUBCORES + lax.axis_index("s")
    tok0 = wid * TPW

    lanes = lax.broadcasted_iota(jnp.int32, (16,), 0)
    neg_inf = jnp.full((16,), -jnp.inf, jnp.float32)
    f32_zero = jnp.zeros((16,), jnp.float32)
    i32_zero = jnp.zeros((16,), jnp.int32)

    @pl.loop(0, TPW // 128)
    def _stage_chunk(c):
        pltpu.sync_copy(x_hbm.at[pl.ds(tok0 + c * 128, 128)], stage)

        @pl.loop(0, 128, unroll=4)
        def _relayout(t):
            dst = (c * 128 + t) * STRIDE
            vs = [stage[t, pl.ds(16 * k, 16)] for k in range(4)]
            for k in range(4):
                plsc.store_scatter(buf, [lanes + (dst + 16 * k)], vs[k])

    @pl.loop(0, GROUPS // 8)
    def _chunk(chunk):
      @pl.loop(0, 8)
      def _group(gi):
        g = gi + chunk * 8
        rowoff = lanes * STRIDE + g * (16 * STRIDE)

        def _max_pass(e, carry):
            m, idx = carry
            v = plsc.load_gather(buf, [rowoff + e])
            gt = v > m
            return jnp.where(gt, v, m), jnp.where(gt, e, idx)

        m, idx = lax.fori_loop(0, NUM_EXPERTS, _max_pass,
                               (neg_inf, i32_zero), unroll=8)

        one = jnp.full((16,), 1.0, jnp.float32)
        for i in range(TOP_K):
            plsc.store_scatter(buf, [rowoff + idx], neg_inf)

            def _fused_pass(e, carry, m=m):
                denom, m2, idx2 = carry
                v = plsc.load_gather(buf, [rowoff + e])
                gap = v - m
                neg_thresh = NEG_TWO_EPS * jnp.maximum(jnp.abs(v), m)
                drop = gap < neg_thresh
                term = jnp.where(drop, f32_zero, jnp.exp(gap))
                gt = v > m2
                return (denom + term,
                        jnp.where(gt, v, m2),
                        jnp.where(gt, e, idx2))

            denom, m2, idx2 = lax.fori_loop(
                0, NUM_EXPERTS, _fused_pass,
                (one, neg_inf, i32_zero), unroll=8)

            orow = gi * 16 + lanes
            ocol = jnp.full((16,), i, jnp.int32)
            plsc.store_scatter(oi_v, [orow, ocol], idx)
            plsc.store_scatter(ov_v, [orow, ocol], jnp.float32(1.0) / denom)
            m, idx = m2, idx2

      base = tok0 + chunk * 128
      pltpu.sync_copy(oi_v, oi_hbm.at[pl.ds(base, 128)])
      pltpu.sync_copy(ov_v, ov_hbm.at[pl.ds(base, 128)])


@jax.jit
def kernel(router_logits):
    x = router_logits.astype(jnp.float32)
    mesh = plsc.VectorSubcoreMesh(
        core_axis_name="c", subcore_axis_name="s",
        num_cores=NUM_CORES, num_subcores=NUM_SUBCORES)
    run = pl.kernel(
        _routing_body,
        out_type=[
            jax.ShapeDtypeStruct((NUM_TOKENS, TOP_K), jnp.int32),
            jax.ShapeDtypeStruct((NUM_TOKENS, TOP_K), jnp.float32),
        ],
        mesh=mesh,
        compiler_params=pltpu.CompilerParams(
            needs_layout_passes=False, use_tc_tiling_on_sc=True),
        scratch_types=[
            pltpu.VMEM((TPW * STRIDE,), jnp.float32),
            pltpu.VMEM((128, NUM_EXPERTS), jnp.float32),
            pltpu.VMEM((128, TOP_K), jnp.int32),
            pltpu.VMEM((128, TOP_K), jnp.float32),
        ],
    )
    idxs, vals = run(x)
    return idxs, vals
